# baseline (speedup 1.0000x reference)
import functools

import jax
import jax.numpy as jnp
from jax import lax
from jax.experimental import pallas as pl
from jax.experimental.pallas import tpu as pltpu
from jax.experimental.pallas import tpu_sc as plsc

VOCAB = 1000000
DIM = 64
B = 16384
K = 20

_info = plsc.get_sparse_core_info()
NC, NS = _info.num_cores, _info.num_subcores
NW = NC * NS
BPW = B // NW
G = 32
NG = BPW // G
NCH = (G * K) // 128


def _sc_body(tgt, ctx, negf, emb, cemb, dots_out,
             idx_t, idx_c, idx_n, tb, cb, nb, dots_v, sem0, sem1):
    wid = lax.axis_index("s") * NC + lax.axis_index("c")
    base = wid * BPW
    sems = (sem0, sem1)

    pltpu.sync_copy(tgt.at[pl.ds(base, BPW)], idx_t)
    pltpu.sync_copy(ctx.at[pl.ds(base, BPW)], idx_c)
    pltpu.sync_copy(negf.at[pl.ds(base * K, BPW * K)], idx_n)

    def fire(g):
        p = g % 2
        descs = [
            pltpu.async_copy(emb.at[idx_t.at[pl.ds(g * G, G)]],
                             tb.at[p], sems[p]),
            pltpu.async_copy(cemb.at[idx_c.at[pl.ds(g * G, G)]],
                             cb.at[p], sems[p]),
        ]
        for j in range(NCH):
            descs.append(
                pltpu.async_copy(
                    cemb.at[idx_n.at[pl.ds(g * G * K + j * 128, 128)]],
                    nb.at[p, pl.ds(j * 128, 128)], sems[p]))
        return descs

    def compute(g):
        p = g % 2
        tbp, cbp, nbp = tb.at[p], cb.at[p], nb.at[p]
        for s in range(G // 16):
            rows = s * 16 + lax.iota(jnp.int32, 16)
            nrows = [rows * K + k for k in range(K)]

            def d_body(d, carry):
                dd = jnp.full((16,), d, jnp.int32)
                tvec = plsc.load_gather(tbp, [rows, dd])
                cvec = plsc.load_gather(cbp, [rows, dd])
                out = [carry[0] + tvec * cvec]
                for k in range(K):
                    nvec = plsc.load_gather(nbp, [nrows[k], dd])
                    out.append(carry[1 + k] + tvec * nvec)
                return tuple(out)

            init = tuple(jnp.zeros((16,), jnp.float32) for _ in range(K + 1))
            accs = lax.fori_loop(0, DIM, d_body, init)
            off = g * G + s * 16
            for j in range(K + 1):
                dots_v[j, pl.ds(off, 16)] = accs[j]

    descs = fire(0)
    for g in range(NG):
        nxt = fire(g + 1) if g + 1 < NG else []
        for d in descs:
            d.wait()
        compute(g)
        descs = nxt

    pltpu.sync_copy(dots_v, dots_out.at[:, pl.ds(base, BPW)])


@functools.partial(
    pl.kernel,
    out_type=jax.ShapeDtypeStruct((K + 1, B), jnp.float32),
    mesh=plsc.VectorSubcoreMesh(core_axis_name="c", subcore_axis_name="s"),
    scratch_types=[
        pltpu.VMEM((BPW,), jnp.int32),
        pltpu.VMEM((BPW,), jnp.int32),
        pltpu.VMEM((BPW * K,), jnp.int32),
        pltpu.VMEM((2, G, DIM), jnp.float32),
        pltpu.VMEM((2, G, DIM), jnp.float32),
        pltpu.VMEM((2, G * K, DIM), jnp.float32),
        pltpu.VMEM((K + 1, BPW), jnp.float32),
        pltpu.SemaphoreType.DMA,
        pltpu.SemaphoreType.DMA,
    ],
    compiler_params=pltpu.CompilerParams(use_tc_tiling_on_sc=False,
                                         needs_layout_passes=False),
)
def _sc_dots(tgt, ctx, negf, emb, cemb, dots_out,
             idx_t, idx_c, idx_n, tb, cb, nb, dots_v, sem0, sem1):
    _sc_body(tgt, ctx, negf, emb, cemb, dots_out,
             idx_t, idx_c, idx_n, tb, cb, nb, dots_v, sem0, sem1)


BB = 2048


def _log_sigmoid(x):
    return jnp.where(x >= 0, -jnp.log1p(jnp.exp(-x)), x - jnp.log1p(jnp.exp(x)))


def _tc_loss_kernel(d_ref, out_ref):
    x = d_ref[...]
    pos = x[0:1, :]
    neg = x[1:, :]
    total = jnp.sum(_log_sigmoid(pos)) + jnp.sum(_log_sigmoid(-neg))

    @pl.when(pl.program_id(0) == 0)
    def _init():
        out_ref[...] = jnp.zeros_like(out_ref)

    out_ref[...] += jnp.reshape(total, (1, 1))


def _tc_loss(dots):
    return pl.pallas_call(
        _tc_loss_kernel,
        grid=(B // BB,),
        in_specs=[pl.BlockSpec((K + 1, BB), lambda i: (0, i))],
        out_specs=pl.BlockSpec((1, 1), lambda i: (0, 0)),
        out_shape=jax.ShapeDtypeStruct((1, 1), jnp.float32),
    )(dots)


def kernel(target, context, negative_samples, embeddings, context_embeddings):
    tgt = target.astype(jnp.int32)
    ctx = context.astype(jnp.int32)
    negf = negative_samples.astype(jnp.int32).reshape(-1)

    dots = _sc_dots(tgt, ctx, negf, embeddings, context_embeddings)
    acc = _tc_loss(dots)
    return -acc[0, 0] / B


# speedup vs baseline: 1.2585x; 1.2585x over previous
import functools

import jax
import jax.numpy as jnp
from jax import lax
from jax.experimental import pallas as pl
from jax.experimental.pallas import tpu as pltpu
from jax.experimental.pallas import tpu_sc as plsc

VOCAB = 1000000
HV = VOCAB // 2
DIM = 64
WD = 128
B = 16384
K = 20

_info = plsc.get_sparse_core_info()
NC, NS = _info.num_cores, _info.num_subcores
NW = NC * NS
BPW = B // NW
G = 16
NG = BPW // G
CH = 80
NCH = (G * K) // CH


def _log_sigmoid_vec(x):
    u = jnp.exp(-jnp.abs(x))
    z = u / (2.0 + u)
    z2 = z * z
    p = 1.0 / 11.0
    p = p * z2 + 1.0 / 9.0
    p = p * z2 + 1.0 / 7.0
    p = p * z2 + 1.0 / 5.0
    p = p * z2 + 1.0 / 3.0
    p = p * z2 + 1.0
    return jnp.minimum(x, 0.0) - 2.0 * z * p


def _sc_body(tr, th, cr, ch_, nr, nh, emb, cemb, out,
             idx_tr, idx_th, idx_cr, idx_ch, idx_nr, idx_nh,
             tb, cb, nb, part_v, sem0, sem1):
    wid = lax.axis_index("s") * NC + lax.axis_index("c")
    base = wid * BPW
    sems = (sem0, sem1)

    pltpu.sync_copy(tr.at[pl.ds(base, BPW)], idx_tr)
    pltpu.sync_copy(th.at[pl.ds(base, BPW)], idx_th)
    pltpu.sync_copy(cr.at[pl.ds(base, BPW)], idx_cr)
    pltpu.sync_copy(ch_.at[pl.ds(base, BPW)], idx_ch)
    pltpu.sync_copy(nr.at[pl.ds(base * K, BPW * K)], idx_nr)
    pltpu.sync_copy(nh.at[pl.ds(base * K, BPW * K)], idx_nh)

    def descs(g, p):
        ds_ = [
            pltpu.make_async_copy(emb.at[idx_tr.at[pl.ds(g * G, G)]],
                                  tb.at[p], sems[p]),
            pltpu.make_async_copy(cemb.at[idx_cr.at[pl.ds(g * G, G)]],
                                  cb.at[p], sems[p]),
        ]
        for j in range(NCH):
            ds_.append(
                pltpu.make_async_copy(
                    cemb.at[idx_nr.at[pl.ds(g * G * K + j * CH, CH)]],
                    nb.at[p, pl.ds(j * CH, CH)], sems[p]))
        return ds_

    def fire(g, p):
        for d_ in descs(g, p):
            d_.start()

    def drain(g, p):
        for d_ in descs(g, p):
            d_.wait()

    def compute(g, p):
        tbp, cbp, nbp = tb.at[p], cb.at[p], nb.at[p]
        rows = lax.iota(jnp.int32, 16)
        nrows = [rows * K + k for k in range(K)]
        goff = g * G
        th16 = idx_th[pl.ds(goff, 16)]
        ch16 = idx_ch[pl.ds(goff, 16)]
        nh16 = [plsc.load_gather(idx_nh, [nrows[k] + g * (G * K)])
                for k in range(K)]

        def d_body(d, carry):
            ddrot = (jnp.full((16,), d, jnp.int32) + rows) & (DIM - 1)
            tvec = plsc.load_gather(tbp, [rows, th16 + ddrot])
            cvec = plsc.load_gather(cbp, [rows, ch16 + ddrot])
            out_ = [carry[0] + tvec * cvec]
            for k in range(K):
                nvec = plsc.load_gather(nbp, [nrows[k], nh16[k] + ddrot])
                out_.append(carry[1 + k] + tvec * nvec)
            return tuple(out_)

        init = tuple(jnp.zeros((16,), jnp.float32) for _ in range(K + 1))
        accs = lax.fori_loop(0, DIM, d_body, init)
        a = part_v[...] + _log_sigmoid_vec(accs[0])
        for j in range(1, K + 1):
            a = a + _log_sigmoid_vec(-accs[j])
        part_v[...] = a

    part_v[...] = jnp.zeros((16,), jnp.float32)
    fire(0, 0)
    fire(1, 1)

    def g_body(gg, _):
        g0 = 2 * gg
        drain(g0, 0)
        compute(g0, 0)
        fire(g0 + 2, 0)
        drain(g0 + 1, 1)
        compute(g0 + 1, 1)
        fire(g0 + 3, 1)
        return 0

    lax.fori_loop(0, NG // 2 - 1, g_body, 0)
    drain(NG - 2, 0)
    compute(NG - 2, 0)
    drain(NG - 1, 1)
    compute(NG - 1, 1)

    pltpu.sync_copy(part_v, out.at[pl.ds(wid * 16, 16)])


@functools.partial(
    pl.kernel,
    out_type=jax.ShapeDtypeStruct((NW * 16,), jnp.float32),
    mesh=plsc.VectorSubcoreMesh(core_axis_name="c", subcore_axis_name="s"),
    scratch_types=[
        pltpu.VMEM((BPW,), jnp.int32),
        pltpu.VMEM((BPW,), jnp.int32),
        pltpu.VMEM((BPW,), jnp.int32),
        pltpu.VMEM((BPW,), jnp.int32),
        pltpu.VMEM((BPW * K,), jnp.int32),
        pltpu.VMEM((BPW * K,), jnp.int32),
        pltpu.VMEM((2, G, WD), jnp.float32),
        pltpu.VMEM((2, G, WD), jnp.float32),
        pltpu.VMEM((2, G * K, WD), jnp.float32),
        pltpu.VMEM((16,), jnp.float32),
        pltpu.SemaphoreType.DMA,
        pltpu.SemaphoreType.DMA,
    ],
    compiler_params=pltpu.CompilerParams(use_tc_tiling_on_sc=True,
                                         needs_layout_passes=False),
)
def _sc_loss(tr, th, cr, ch_, nr, nh, emb, cemb, out,
             idx_tr, idx_th, idx_cr, idx_ch, idx_nr, idx_nh,
             tb, cb, nb, part_v, sem0, sem1):
    _sc_body(tr, th, cr, ch_, nr, nh, emb, cemb, out,
             idx_tr, idx_th, idx_cr, idx_ch, idx_nr, idx_nh,
             tb, cb, nb, part_v, sem0, sem1)


def kernel(target, context, negative_samples, embeddings, context_embeddings):
    tgt = target.astype(jnp.int32)
    ctx = context.astype(jnp.int32)
    negf = negative_samples.astype(jnp.int32).reshape(-1)

    parts = _sc_loss(tgt >> 1, (tgt & 1) * DIM,
                     ctx >> 1, (ctx & 1) * DIM,
                     negf >> 1, (negf & 1) * DIM,
                     embeddings.reshape(HV, WD),
                     context_embeddings.reshape(HV, WD))
    return -jnp.sum(parts) / B


# speedup vs baseline: 1.8926x; 1.5039x over previous
import functools

import jax
import jax.numpy as jnp
from jax import lax
from jax.experimental import pallas as pl
from jax.experimental.pallas import tpu as pltpu
from jax.experimental.pallas import tpu_sc as plsc

VOCAB = 1000000
HV = VOCAB // 2
DIM = 64
WD = 128
B = 16384
K = 20

_info = plsc.get_sparse_core_info()
NC, NS = _info.num_cores, _info.num_subcores
NW = NC * NS
BPW = B // NW
G = 16
NG = BPW // G
CH = 80
NCH = (G * K) // CH


def _log_sigmoid_vec(x):
    u = jnp.exp(-jnp.abs(x))
    z = u / (2.0 + u)
    z2 = z * z
    p = 1.0 / 11.0
    p = p * z2 + 1.0 / 9.0
    p = p * z2 + 1.0 / 7.0
    p = p * z2 + 1.0 / 5.0
    p = p * z2 + 1.0 / 3.0
    p = p * z2 + 1.0
    return jnp.minimum(x, 0.0) - 2.0 * z * p


def _sc_body(tr, th, cr, ch_, nr, nh, emb, cemb, out,
             idx_tr, idx_th, idx_cr, idx_ch, idx_nr, idx_nh,
             tb, cb, nb, part_v, sem0, sem1):
    wid = lax.axis_index("s") * NC + lax.axis_index("c")
    base = wid * BPW
    sems = (sem0, sem1)

    pltpu.sync_copy(tr.at[pl.ds(base, BPW)], idx_tr)
    pltpu.sync_copy(th.at[pl.ds(base, BPW)], idx_th)
    pltpu.sync_copy(cr.at[pl.ds(base, BPW)], idx_cr)
    pltpu.sync_copy(ch_.at[pl.ds(base, BPW)], idx_ch)
    pltpu.sync_copy(nr.at[pl.ds(base * K, BPW * K)], idx_nr)
    pltpu.sync_copy(nh.at[pl.ds(base * K, BPW * K)], idx_nh)

    def descs(g, p):
        ds_ = [
            pltpu.make_async_copy(emb.at[idx_tr.at[pl.ds(g * G, G)]],
                                  tb.at[p], sems[p]),
            pltpu.make_async_copy(cemb.at[idx_cr.at[pl.ds(g * G, G)]],
                                  cb.at[p], sems[p]),
        ]
        for j in range(NCH):
            ds_.append(
                pltpu.make_async_copy(
                    cemb.at[idx_nr.at[pl.ds(g * G * K + j * CH, CH)]],
                    nb.at[p, pl.ds(j * CH, CH)], sems[p]))
        return ds_

    def fire(g, p):
        for d_ in descs(g, p):
            d_.start()

    def drain(g, p):
        for d_ in descs(g, p):
            d_.wait()

    def compute(g, p):
        tbp, cbp, nbp = tb.at[p], cb.at[p], nb.at[p]
        rows = lax.iota(jnp.int32, 16)
        nrows = [rows * K + k for k in range(K)]
        goff = g * G
        th16 = idx_th[pl.ds(goff, 16)]
        ch16 = idx_ch[pl.ds(goff, 16)]
        nh16 = [plsc.load_gather(idx_nh, [nrows[k] + g * (G * K)])
                for k in range(K)]

        def d_body(d, carry):
            ddrot = (jnp.full((16,), d, jnp.int32) + rows) & (DIM - 1)
            tvec = plsc.load_gather(tbp, [rows, th16 + ddrot])
            cvec = plsc.load_gather(cbp, [rows, ch16 + ddrot])
            out_ = [carry[0] + tvec * cvec]
            for k in range(K):
                nvec = plsc.load_gather(nbp, [nrows[k], nh16[k] + ddrot])
                out_.append(carry[1 + k] + tvec * nvec)
            return tuple(out_)

        init = tuple(jnp.zeros((16,), jnp.float32) for _ in range(K + 1))
        accs = lax.fori_loop(0, DIM, d_body, init)
        a = part_v[...] + _log_sigmoid_vec(accs[0])
        for j in range(1, K + 1):
            a = a + _log_sigmoid_vec(-accs[j])
        part_v[...] = a

    part_v[...] = jnp.zeros((16,), jnp.float32)
    fire(0, 0)
    fire(1, 1)

    def g_body(gg, _):
        g0 = 2 * gg
        drain(g0, 0)
        compute(g0, 0)
        fire(g0 + 2, 0)
        drain(g0 + 1, 1)
        compute(g0 + 1, 1)
        fire(g0 + 3, 1)
        return 0

    lax.fori_loop(0, NG // 2 - 1, g_body, 0)
    drain(NG - 2, 0)
    compute(NG - 2, 0)
    drain(NG - 1, 1)
    compute(NG - 1, 1)

    pltpu.sync_copy(part_v, out.at[pl.ds(wid * 16, 16)])


@functools.partial(
    pl.kernel,
    out_type=jax.ShapeDtypeStruct((NW * 16,), jnp.float32),
    mesh=plsc.VectorSubcoreMesh(core_axis_name="c", subcore_axis_name="s"),
    scratch_types=[
        pltpu.VMEM((BPW,), jnp.int32),
        pltpu.VMEM((BPW,), jnp.int32),
        pltpu.VMEM((BPW,), jnp.int32),
        pltpu.VMEM((BPW,), jnp.int32),
        pltpu.VMEM((BPW * K,), jnp.int32),
        pltpu.VMEM((BPW * K,), jnp.int32),
        pltpu.VMEM((2, G, WD), jnp.float32),
        pltpu.VMEM((2, G, WD), jnp.float32),
        pltpu.VMEM((2, G * K, WD), jnp.float32),
        pltpu.VMEM((16,), jnp.float32),
        pltpu.SemaphoreType.DMA,
        pltpu.SemaphoreType.DMA,
    ],
    compiler_params=pltpu.CompilerParams(use_tc_tiling_on_sc=True,
                                         needs_layout_passes=False),
)
def _sc_loss(tr, th, cr, ch_, nr, nh, emb, cemb, out,
             idx_tr, idx_th, idx_cr, idx_ch, idx_nr, idx_nh,
             tb, cb, nb, part_v, sem0, sem1):
    _sc_body(tr, th, cr, ch_, nr, nh, emb, cemb, out,
             idx_tr, idx_th, idx_cr, idx_ch, idx_nr, idx_nh,
             tb, cb, nb, part_v, sem0, sem1)


CBLK = 2048
RBLK = CBLK // 2
NBO = -(-VOCAB // CBLK)
HV2 = NBO * RBLK


def _tc_pair_xpose_kernel(a_ref, b_ref, ao_ref, bo_ref):
    a = a_ref[...]
    ao_ref[...] = jnp.concatenate([a[:, :RBLK].T, a[:, RBLK:].T], axis=1)
    b = b_ref[...]
    bo_ref[...] = jnp.concatenate([b[:, :RBLK].T, b[:, RBLK:].T], axis=1)


def _tc_pair_xpose(embT, cembT):
    out = jax.ShapeDtypeStruct((HV2, WD), jnp.float32)
    return pl.pallas_call(
        _tc_pair_xpose_kernel,
        grid=(NBO,),
        in_specs=[pl.BlockSpec((DIM, CBLK), lambda i: (0, i)),
                  pl.BlockSpec((DIM, CBLK), lambda i: (0, i))],
        out_specs=[pl.BlockSpec((RBLK, WD), lambda i: (i, 0)),
                   pl.BlockSpec((RBLK, WD), lambda i: (i, 0))],
        out_shape=(out, out),
    )(embT, cembT)


def kernel(target, context, negative_samples, embeddings, context_embeddings):
    tgt = target.astype(jnp.int32)
    ctx = context.astype(jnp.int32)
    negf = negative_samples.astype(jnp.int32).reshape(-1)

    emb2, cemb2 = _tc_pair_xpose(embeddings.T, context_embeddings.T)

    def prow(v):
        return ((v >> 11) << 10) + (v & 1023)

    def phalf(v):
        return ((v >> 10) & 1) * DIM

    parts = _sc_loss(prow(tgt), phalf(tgt), prow(ctx), phalf(ctx),
                     prow(negf), phalf(negf), emb2, cemb2)
    return -jnp.sum(parts) / B
